# baseline (speedup 1.0000x reference)
import functools

import jax
import jax.numpy as jnp
from jax import lax
from jax.experimental import pallas as pl
from jax.experimental.pallas import tpu as pltpu
from jax.experimental.pallas import tpu_sc as plsc

N = 10000
E = 320000
D = 128
L = 4
D_OUT = 51
EPS = 1e-5

NC = 2
NS = 16
NW = NC * NS
EW = E // NW
B = 80
SUP = 25
NSUPER = EW // (B * SUP)
NPAD = 10112
RPS = NPAD // NS


def _bn(h, g, b):
    mu = jnp.mean(h, axis=0, keepdims=True)
    hc = h - mu
    var = jnp.mean(hc * hc, axis=0, keepdims=True)
    return g * hc * lax.rsqrt(var + EPS) + b


def _post(a, g, b):
    o = jax.nn.relu(_bn(a, g, b))
    nrm = jnp.sqrt(jnp.sum(o * o, axis=-1, keepdims=True))
    return o / (nrm + 1e-12)


def _proj(h, w, b):
    return jnp.dot(h, w, preferred_element_type=jnp.float32) + b


def _pre_body(x, wpre, bpre, gpre, bepre, wkn, wq, wv, ws, bkn, bq, bv, bs,
              h_o, kn_o, qv_o, s_o):
    h = _proj(x[...], wpre[...], bpre[...])
    h = _post(h, gpre[...], bepre[...])
    h_o[...] = h
    kn_o[...] = _proj(h, wkn[...], bkn[...])
    qv_o[:, 0:D] = _proj(h, wq[...], bq[...])
    qv_o[:, D:2 * D] = _proj(h, wv[...], bv[...])
    s_o[...] = _proj(h, ws[...], bs[...])


_kqv_shapes = (
    jax.ShapeDtypeStruct((N, D), jnp.float32),
    jax.ShapeDtypeStruct((N, D), jnp.float32),
    jax.ShapeDtypeStruct((N, 2 * D), jnp.float32),
    jax.ShapeDtypeStruct((N, D), jnp.float32),
)

_pre_call = pl.pallas_call(_pre_body, out_shape=_kqv_shapes)


def _mid_body(aggr, h, s, g, be, wkn, wq, wv, ws, bkn, bq, bv, bs,
              h_o, kn_o, qv_o, s_o):
    a = aggr[0:N, :] + aggr[NPAD:NPAD + N, :] + s[...]
    h2 = h[...] + _post(a, g[...], be[...])
    h_o[...] = h2
    kn_o[...] = _proj(h2, wkn[...], bkn[...])
    qv_o[:, 0:D] = _proj(h2, wq[...], bq[...])
    qv_o[:, D:2 * D] = _proj(h2, wv[...], bv[...])
    s_o[...] = _proj(h2, ws[...], bs[...])


_mid_call = pl.pallas_call(_mid_body, out_shape=_kqv_shapes)


def _final_body(aggr, h, s, g, be, wh1, bh1, wh2, bh2, pred_o):
    a = aggr[0:N, :] + aggr[NPAD:NPAD + N, :] + s[...]
    h2 = h[...] + _post(a, g[...], be[...])
    z = jax.nn.relu(_proj(h2, wh1[...], bh1[...]))
    pred_o[...] = _proj(z, wh2[...], bh2[...])


_final_call = pl.pallas_call(
    _final_body,
    out_shape=jax.ShapeDtypeStruct((N, D_OUT), jnp.float32),
)


_sc_mesh = plsc.VectorSubcoreMesh(core_axis_name="c", subcore_axis_name="s")


@functools.partial(
    pl.kernel,
    out_type=jax.ShapeDtypeStruct((2 * NPAD, D), jnp.float32),
    mesh=_sc_mesh,
    scratch_types=[
        pltpu.VMEM((SUP, B), jnp.int32),
        pltpu.VMEM((SUP, B), jnp.int32),
        pltpu.VMEM((B,), jnp.int32),
        pltpu.VMEM((B,), jnp.int32),
        pltpu.VMEM((B, D), jnp.float32),
        pltpu.VMEM((B, 2 * D), jnp.float32),
        pltpu.VMEM_SHARED((NPAD, D), jnp.float32),
        pltpu.SemaphoreType.DMA,
        pltpu.SemaphoreType.DMA,
    ],
)
def _edge_call(kn_hbm, qv_hbm, src_hbm, dst_hbm, out_hbm,
               srcv, dstv, srcj, dstj, knr, qvr, aggr, sem1, sem2):
    cid = lax.axis_index("c")
    sid = lax.axis_index("s")
    wid = sid * NC + cid
    base = wid * NSUPER

    zeros16 = jnp.zeros((16,), jnp.float32)

    def zrow(r, _):
        for t in range(D // 16):
            knr[r, pl.ds(t * 16, 16)] = zeros16
        return 0

    lax.fori_loop(0, 8, zrow, 0)

    def zcopy(t, _):
        pltpu.sync_copy(knr.at[pl.ds(0, 8)],
                        aggr.at[pl.ds(sid * RPS + t * 8, 8)])
        return 0

    lax.fori_loop(0, RPS // 8, zcopy, 0)
    plsc.subcore_barrier()

    def chunk(j, _):
        for t in range(B // 16):
            sl = pl.ds(t * 16, 16)
            srcj[sl] = srcv[j, sl]
            dstj[sl] = dstv[j, sl]
        c1 = pltpu.async_copy(kn_hbm.at[dstj], knr, sem1)
        c2 = pltpu.async_copy(qv_hbm.at[srcj], qvr, sem2)
        c1.wait()
        c2.wait()

        def row(r, _):
            for t in range(D // 16):
                sl = pl.ds(t * 16, 16)
                ex = jnp.exp(knr[r, sl] - qvr[r, sl])
                knr[r, sl] = qvr[r, pl.ds(D + t * 16, 16)] / (1.0 + ex)
            return 0

        lax.fori_loop(0, B, row, 0)
        pltpu.sync_copy(knr, aggr.at[dstj], add=True)
        return 0

    def super_body(g, _):
        pltpu.sync_copy(src_hbm.at[base + g], srcv)
        pltpu.sync_copy(dst_hbm.at[base + g], dstv)
        lax.fori_loop(0, SUP, chunk, 0)
        return 0

    lax.fori_loop(0, NSUPER, super_body, 0)
    plsc.subcore_barrier()

    r0 = sid * RPS
    pltpu.sync_copy(aggr.at[pl.ds(r0, RPS)],
                    out_hbm.at[pl.ds(cid * NPAD + r0, RPS)])


def kernel(x, edge_index, W_pre, b_pre, g_pre, be_pre, Wk, Wq, Wv, Ws,
           bk, bq, bv, bs, gamma, beta, Wh1, bh1, Wh2, bh2):
    src = edge_index[0].reshape(NW * NSUPER, SUP, B)
    dst = edge_index[1].reshape(NW * NSUPER, SUP, B)
    r = lambda t: t.reshape(1, -1)

    h, kn, qv, s = _pre_call(
        x, W_pre, r(b_pre), r(g_pre), r(be_pre),
        -Wk[0], Wq[0], Wv[0], Ws[0], r(-bk[0]), r(bq[0]), r(bv[0]), r(bs[0]))

    for l in range(L):
        aggr2 = _edge_call(kn, qv, src, dst)
        if l < L - 1:
            h, kn, qv, s = _mid_call(
                aggr2, h, s, r(gamma[l]), r(beta[l]),
                -Wk[l + 1], Wq[l + 1], Wv[l + 1], Ws[l + 1],
                r(-bk[l + 1]), r(bq[l + 1]), r(bv[l + 1]), r(bs[l + 1]))
        else:
            pred = _final_call(
                aggr2, h, s, r(gamma[l]), r(beta[l]),
                Wh1, r(bh1), Wh2, r(bh2))
    return pred


# speedup vs baseline: 3.4376x; 3.4376x over previous
import functools

import jax
import jax.numpy as jnp
from jax import lax
from jax.experimental import pallas as pl
from jax.experimental.pallas import tpu as pltpu
from jax.experimental.pallas import tpu_sc as plsc

N = 10000
E = 320000
D = 128
L = 4
D_OUT = 51
EPS = 1e-5

NC = 2
NS = 16
NW = NC * NS
EW = E // NW
B = 80
SUP = 25
NSUPER = EW // (B * SUP)
NPAD = 10112
RPS = NPAD // NS


def _bn(h, g, b):
    mu = jnp.mean(h, axis=0, keepdims=True)
    hc = h - mu
    var = jnp.mean(hc * hc, axis=0, keepdims=True)
    return g * hc * lax.rsqrt(var + EPS) + b


def _post(a, g, b):
    o = jax.nn.relu(_bn(a, g, b))
    nrm = jnp.sqrt(jnp.sum(o * o, axis=-1, keepdims=True))
    return o / (nrm + 1e-12)


def _proj(h, w, b):
    return jnp.dot(h, w, preferred_element_type=jnp.float32) + b


def _pre_body(x, wpre, bpre, gpre, bepre, wkn, wq, wv, ws, bkn, bq, bv, bs,
              h_o, kn_o, qv_o, s_o):
    h = _proj(x[...], wpre[...], bpre[...])
    h = _post(h, gpre[...], bepre[...])
    h_o[...] = h
    kn_o[...] = _proj(h, wkn[...], bkn[...])
    qv_o[0:N, :] = _proj(h, wq[...], bq[...])
    qv_o[N:2 * N, :] = _proj(h, wv[...], bv[...])
    s_o[...] = _proj(h, ws[...], bs[...])


_kqv_shapes = (
    jax.ShapeDtypeStruct((N, D), jnp.float32),
    jax.ShapeDtypeStruct((N, D), jnp.float32),
    jax.ShapeDtypeStruct((2 * N, D), jnp.float32),
    jax.ShapeDtypeStruct((N, D), jnp.float32),
)

_pre_call = pl.pallas_call(_pre_body, out_shape=_kqv_shapes)


def _mid_body(aggr, h, s, g, be, wkn, wq, wv, ws, bkn, bq, bv, bs,
              h_o, kn_o, qv_o, s_o):
    a = aggr[0:N, :] + aggr[NPAD:NPAD + N, :] + s[...]
    h2 = h[...] + _post(a, g[...], be[...])
    h_o[...] = h2
    kn_o[...] = _proj(h2, wkn[...], bkn[...])
    qv_o[0:N, :] = _proj(h2, wq[...], bq[...])
    qv_o[N:2 * N, :] = _proj(h2, wv[...], bv[...])
    s_o[...] = _proj(h2, ws[...], bs[...])


_mid_call = pl.pallas_call(_mid_body, out_shape=_kqv_shapes)


def _final_body(aggr, h, s, g, be, wh1, bh1, wh2, bh2, pred_o):
    a = aggr[0:N, :] + aggr[NPAD:NPAD + N, :] + s[...]
    h2 = h[...] + _post(a, g[...], be[...])
    z = jax.nn.relu(_proj(h2, wh1[...], bh1[...]))
    pred_o[...] = _proj(z, wh2[...], bh2[...])


_final_call = pl.pallas_call(
    _final_body,
    out_shape=jax.ShapeDtypeStruct((N, D_OUT), jnp.float32),
)


_sc_mesh = plsc.VectorSubcoreMesh(core_axis_name="c", subcore_axis_name="s")


@functools.partial(
    pl.kernel,
    out_type=jax.ShapeDtypeStruct((2 * NPAD, D), jnp.float32),
    mesh=_sc_mesh,
    scratch_types=[
        pltpu.VMEM((SUP, B), jnp.int32),
        pltpu.VMEM((SUP, B), jnp.int32),
        pltpu.VMEM((B,), jnp.int32),
        pltpu.VMEM((B,), jnp.int32),
        pltpu.VMEM((B,), jnp.int32),
        pltpu.VMEM((B, D), jnp.float32),
        pltpu.VMEM((B, D), jnp.float32),
        pltpu.VMEM((B, D), jnp.float32),
        pltpu.VMEM_SHARED((NPAD, D), jnp.float32),
        pltpu.SemaphoreType.DMA,
        pltpu.SemaphoreType.DMA,
    ],
)
def _edge_call(kn_hbm, qv_hbm, src_hbm, dst_hbm, out_hbm,
               srcv, dstv, srcj, srcjn, dstj, knr, qr, vr, aggr, sem1, sem2):
    cid = lax.axis_index("c")
    sid = lax.axis_index("s")
    wid = sid * NC + cid
    base = wid * NSUPER

    zeros16 = jnp.zeros((16,), jnp.float32)

    def zrow(r, _):
        for t in range(D // 16):
            knr[r, pl.ds(t * 16, 16)] = zeros16
        return 0

    lax.fori_loop(0, 8, zrow, 0)

    def zcopy(t, _):
        pltpu.sync_copy(knr.at[pl.ds(0, 8)],
                        aggr.at[pl.ds(sid * RPS + t * 8, 8)])
        return 0

    lax.fori_loop(0, RPS // 8, zcopy, 0)
    plsc.subcore_barrier()

    def chunk(j, _):
        nvec = jnp.zeros((16,), jnp.int32) + N
        for t in range(B // 16):
            sl = pl.ds(t * 16, 16)
            sj = srcv[j, sl]
            srcj[sl] = sj
            srcjn[sl] = sj + nvec
            dstj[sl] = dstv[j, sl]
        c1 = pltpu.async_copy(kn_hbm.at[dstj], knr, sem1)
        c2 = pltpu.async_copy(qv_hbm.at[srcj], qr, sem2)
        c3 = pltpu.async_copy(qv_hbm.at[srcjn], vr, sem1)
        c1.wait()
        c2.wait()
        c3.wait()

        def row(r, _):
            for t in range(D // 16):
                sl = pl.ds(t * 16, 16)
                ex = jnp.exp(knr[r, sl] - qr[r, sl])
                knr[r, sl] = vr[r, sl] / (1.0 + ex)
            return 0

        lax.fori_loop(0, B, row, 0)
        pltpu.sync_copy(knr, aggr.at[dstj], add=True)
        return 0

    def super_body(g, _):
        pltpu.sync_copy(src_hbm.at[base + g], srcv)
        pltpu.sync_copy(dst_hbm.at[base + g], dstv)
        lax.fori_loop(0, SUP, chunk, 0)
        return 0

    lax.fori_loop(0, NSUPER, super_body, 0)
    plsc.subcore_barrier()

    r0 = sid * RPS
    pltpu.sync_copy(aggr.at[pl.ds(r0, RPS)],
                    out_hbm.at[pl.ds(cid * NPAD + r0, RPS)])


def kernel(x, edge_index, W_pre, b_pre, g_pre, be_pre, Wk, Wq, Wv, Ws,
           bk, bq, bv, bs, gamma, beta, Wh1, bh1, Wh2, bh2):
    src = edge_index[0].reshape(NW * NSUPER, SUP, B)
    dst = edge_index[1].reshape(NW * NSUPER, SUP, B)
    r = lambda t: t.reshape(1, -1)

    h, kn, qv, s = _pre_call(
        x, W_pre, r(b_pre), r(g_pre), r(be_pre),
        -Wk[0], Wq[0], Wv[0], Ws[0], r(-bk[0]), r(bq[0]), r(bv[0]), r(bs[0]))

    for l in range(L):
        aggr2 = _edge_call(kn, qv, src, dst)
        if l < L - 1:
            h, kn, qv, s = _mid_call(
                aggr2, h, s, r(gamma[l]), r(beta[l]),
                -Wk[l + 1], Wq[l + 1], Wv[l + 1], Ws[l + 1],
                r(-bk[l + 1]), r(bq[l + 1]), r(bv[l + 1]), r(bs[l + 1]))
        else:
            pred = _final_call(
                aggr2, h, s, r(gamma[l]), r(beta[l]),
                Wh1, r(bh1), Wh2, r(bh2))
    return pred


# speedup vs baseline: 5.0768x; 1.4769x over previous
import functools

import jax
import jax.numpy as jnp
from jax import lax
from jax.experimental import pallas as pl
from jax.experimental.pallas import tpu as pltpu
from jax.experimental.pallas import tpu_sc as plsc

N = 10000
E = 320000
D = 128
L = 4
D_OUT = 51
EPS = 1e-5

NC = 2
NS = 16
NW = NC * NS
EW = E // NW
B = 40
SUP = 50
NSUPER = EW // (B * SUP)
NPAD = 10112
RPS = NPAD // NS


def _bn(h, g, b):
    mu = jnp.mean(h, axis=0, keepdims=True)
    hc = h - mu
    var = jnp.mean(hc * hc, axis=0, keepdims=True)
    return g * hc * lax.rsqrt(var + EPS) + b


def _post(a, g, b):
    o = jax.nn.relu(_bn(a, g, b))
    nrm = jnp.sqrt(jnp.sum(o * o, axis=-1, keepdims=True))
    return o / (nrm + 1e-12)


def _proj(h, w, b):
    return jnp.dot(h, w, preferred_element_type=jnp.float32) + b


def _pre_body(x, wpre, bpre, gpre, bepre, wkn, wq, wv, ws, bkn, bq, bv, bs,
              h_o, kn_o, qv_o, s_o):
    h = _proj(x[...], wpre[...], bpre[...])
    h = _post(h, gpre[...], bepre[...])
    h_o[...] = h
    kn_o[...] = _proj(h, wkn[...], bkn[...])
    qv_o[0:N, :] = _proj(h, wq[...], bq[...])
    qv_o[N:2 * N, :] = _proj(h, wv[...], bv[...])
    s_o[...] = _proj(h, ws[...], bs[...])


_kqv_shapes = (
    jax.ShapeDtypeStruct((N, D), jnp.float32),
    jax.ShapeDtypeStruct((N, D), jnp.float32),
    jax.ShapeDtypeStruct((2 * N, D), jnp.float32),
    jax.ShapeDtypeStruct((N, D), jnp.float32),
)

_pre_call = pl.pallas_call(_pre_body, out_shape=_kqv_shapes)


def _mid_body(aggr, h, s, g, be, wkn, wq, wv, ws, bkn, bq, bv, bs,
              h_o, kn_o, qv_o, s_o):
    a = aggr[0:N, :] + aggr[NPAD:NPAD + N, :] + s[...]
    h2 = h[...] + _post(a, g[...], be[...])
    h_o[...] = h2
    kn_o[...] = _proj(h2, wkn[...], bkn[...])
    qv_o[0:N, :] = _proj(h2, wq[...], bq[...])
    qv_o[N:2 * N, :] = _proj(h2, wv[...], bv[...])
    s_o[...] = _proj(h2, ws[...], bs[...])


_mid_call = pl.pallas_call(_mid_body, out_shape=_kqv_shapes)


def _final_body(aggr, h, s, g, be, wh1, bh1, wh2, bh2, pred_o):
    a = aggr[0:N, :] + aggr[NPAD:NPAD + N, :] + s[...]
    h2 = h[...] + _post(a, g[...], be[...])
    z = jax.nn.relu(_proj(h2, wh1[...], bh1[...]))
    pred_o[...] = _proj(z, wh2[...], bh2[...])


_final_call = pl.pallas_call(
    _final_body,
    out_shape=jax.ShapeDtypeStruct((N, D_OUT), jnp.float32),
)


_sc_mesh = plsc.VectorSubcoreMesh(core_axis_name="c", subcore_axis_name="s")


@functools.partial(
    pl.kernel,
    out_type=jax.ShapeDtypeStruct((2 * NPAD, D), jnp.float32),
    mesh=_sc_mesh,
    scratch_types=[
        pltpu.VMEM((SUP, B), jnp.int32),
        pltpu.VMEM((SUP, B), jnp.int32),
        pltpu.VMEM((B,), jnp.int32),
        pltpu.VMEM((B,), jnp.int32),
        pltpu.VMEM((B,), jnp.int32),
        pltpu.VMEM((B,), jnp.int32),
        pltpu.VMEM((B,), jnp.int32),
        pltpu.VMEM((B,), jnp.int32),
        pltpu.VMEM((B, D), jnp.float32),
        pltpu.VMEM((B, D), jnp.float32),
        pltpu.VMEM((B, D), jnp.float32),
        pltpu.VMEM((B, D), jnp.float32),
        pltpu.VMEM((B, D), jnp.float32),
        pltpu.VMEM((B, D), jnp.float32),
        pltpu.VMEM_SHARED((NPAD, D), jnp.float32),
        pltpu.SemaphoreType.DMA,
        pltpu.SemaphoreType.DMA,
    ],
)
def _edge_call(kn_hbm, qv_hbm, src_hbm, dst_hbm, out_hbm,
               srcv, dstv, sj0, sn0, dj0, sj1, sn1, dj1,
               kn0, q0, v0, kn1, q1, v1, aggr, sem1, sem2):
    cid = lax.axis_index("c")
    sid = lax.axis_index("s")
    wid = sid * NC + cid
    base = wid * NSUPER

    zeros16 = jnp.zeros((16,), jnp.float32)

    def zrow(r, _):
        for t in range(D // 16):
            kn0[r, pl.ds(t * 16, 16)] = zeros16
        return 0

    lax.fori_loop(0, 8, zrow, 0)

    def zcopy(t, _):
        pltpu.sync_copy(kn0.at[pl.ds(0, 8)],
                        aggr.at[pl.ds(sid * RPS + t * 8, 8)])
        return 0

    lax.fori_loop(0, RPS // 8, zcopy, 0)
    plsc.subcore_barrier()

    islots = ((sj0, sn0, dj0), (sj1, sn1, dj1))
    gslots = ((kn0, q0, v0, sem1), (kn1, q1, v1, sem2))
    nvec = jnp.zeros((16,), jnp.int32) + N

    def fill_issue(j, slot):
        sj, sn, dj = islots[slot]
        for start in (0, 16, 24):
            sl = pl.ds(start, 16)
            v16 = srcv[j, sl]
            sj[sl] = v16
            sn[sl] = v16 + nvec
            dj[sl] = dstv[j, sl]
        kn, q, v, sem = gslots[slot]
        pltpu.async_copy(kn_hbm.at[dj], kn, sem)
        pltpu.async_copy(qv_hbm.at[sj], q, sem)
        pltpu.async_copy(qv_hbm.at[sn], v, sem)

    def wait_gathers(slot):
        sj, sn, dj = islots[slot]
        kn, q, v, sem = gslots[slot]
        pltpu.make_async_copy(kn_hbm.at[dj], kn, sem).wait()
        pltpu.make_async_copy(qv_hbm.at[sj], q, sem).wait()
        pltpu.make_async_copy(qv_hbm.at[sn], v, sem).wait()

    def compute_scatter(slot):
        sj, sn, dj = islots[slot]
        kn, q, v, _ = gslots[slot]

        def row(r, _):
            for t in range(D // 16):
                sl = pl.ds(t * 16, 16)
                ex = jnp.exp(kn[r, sl] - q[r, sl])
                kn[r, sl] = v[r, sl] / (1.0 + ex)
            return 0

        lax.fori_loop(0, B, row, 0)
        pltpu.sync_copy(kn, aggr.at[dj], add=True)

    for g in range(NSUPER):
        pltpu.sync_copy(src_hbm.at[base + g], srcv)
        pltpu.sync_copy(dst_hbm.at[base + g], dstv)

        fill_issue(0, 0)
        fill_issue(1, 1)

        def body(t, _):
            for slot in range(2):
                jj = 2 * t + slot
                wait_gathers(slot)
                compute_scatter(slot)
                fill_issue(jj + 2, slot)
            return 0

        lax.fori_loop(0, SUP // 2 - 1, body, 0)
        for slot in range(2):
            wait_gathers(slot)
            compute_scatter(slot)

    plsc.subcore_barrier()

    r0 = sid * RPS
    pltpu.sync_copy(aggr.at[pl.ds(r0, RPS)],
                    out_hbm.at[pl.ds(cid * NPAD + r0, RPS)])


def kernel(x, edge_index, W_pre, b_pre, g_pre, be_pre, Wk, Wq, Wv, Ws,
           bk, bq, bv, bs, gamma, beta, Wh1, bh1, Wh2, bh2):
    src = edge_index[0].reshape(NW * NSUPER, SUP, B)
    dst = edge_index[1].reshape(NW * NSUPER, SUP, B)
    r = lambda t: t.reshape(1, -1)

    h, kn, qv, s = _pre_call(
        x, W_pre, r(b_pre), r(g_pre), r(be_pre),
        -Wk[0], Wq[0], Wv[0], Ws[0], r(-bk[0]), r(bq[0]), r(bv[0]), r(bs[0]))

    for l in range(L):
        aggr2 = _edge_call(kn, qv, src, dst)
        if l < L - 1:
            h, kn, qv, s = _mid_call(
                aggr2, h, s, r(gamma[l]), r(beta[l]),
                -Wk[l + 1], Wq[l + 1], Wv[l + 1], Ws[l + 1],
                r(-bk[l + 1]), r(bq[l + 1]), r(bv[l + 1]), r(bs[l + 1]))
        else:
            pred = _final_call(
                aggr2, h, s, r(gamma[l]), r(beta[l]),
                Wh1, r(bh1), Wh2, r(bh2))
    return pred
